# Initial kernel scaffold; baseline (speedup 1.0000x reference)
import jax
import jax.numpy as jnp
from jax import lax
from jax.experimental import pallas as pl
from jax.experimental.pallas import tpu as pltpu
from jax.experimental.pallas import tpu_sc as plsc

_N = 10000
_NP = 10240
_E = 320000
_NC = 2
_NS = 16
_NW = _NC * _NS
_RPT = _NP // _NS
_CHUNK = 128
_NCHUNK = 80
_EP = _NW * _NCHUNK * _CHUNK
_DH = 128
_DO = 64
_NCLS = 40

_mesh = plsc.VectorSubcoreMesh(core_axis_name="c", subcore_axis_name="s")


def _deg_body(dst3, ones_hbm, zeros_hbm, out, didx_v, ones_v, deg_sp):
    c = lax.axis_index("c")
    s = lax.axis_index("s")
    wid = c * _NS + s
    r0 = s * _RPT
    pltpu.sync_copy(zeros_hbm, deg_sp.at[pl.ds(r0, _RPT)])
    pltpu.sync_copy(ones_hbm, ones_v)
    plsc.subcore_barrier()

    def step(j, carry):
        pltpu.sync_copy(dst3.at[wid, j], didx_v)
        pltpu.sync_copy(ones_v, deg_sp.at[didx_v], add=True)
        return carry

    lax.fori_loop(0, _NCHUNK, step, 0)
    plsc.subcore_barrier()
    pltpu.sync_copy(deg_sp.at[pl.ds(r0, _RPT)], out.at[c, pl.ds(r0, _RPT)])


_deg_call = pl.kernel(
    _deg_body,
    out_type=jax.ShapeDtypeStruct((_NC, _NP, 16), jnp.float32),
    mesh=_mesh,
    scratch_types=[
        pltpu.VMEM((_CHUNK,), jnp.int32),
        pltpu.VMEM((_CHUNK, 16), jnp.float32),
        pltpu.VMEM_SHARED((_NP, 16), jnp.float32),
    ],
)


def _make_agg(width):
    def body(h_hbm, src3, dst3, zeros_hbm, out, sidx_v, didx_v, rows_v, agg_sp, sem):
        c = lax.axis_index("c")
        s = lax.axis_index("s")
        wid = c * _NS + s
        r0 = s * _RPT
        pltpu.sync_copy(zeros_hbm, agg_sp.at[pl.ds(r0, _RPT)])
        plsc.subcore_barrier()

        def step(j, carry):
            pltpu.sync_copy(src3.at[wid, j], sidx_v)
            pltpu.async_copy(h_hbm.at[sidx_v], rows_v, sem).wait()
            pltpu.sync_copy(dst3.at[wid, j], didx_v)
            pltpu.sync_copy(rows_v, agg_sp.at[didx_v], add=True)
            return carry

        lax.fori_loop(0, _NCHUNK, step, 0)
        plsc.subcore_barrier()
        pltpu.sync_copy(agg_sp.at[pl.ds(r0, _RPT)], out.at[c, pl.ds(r0, _RPT)])

    return pl.kernel(
        body,
        out_type=jax.ShapeDtypeStruct((_NC, _NP, width), jnp.float32),
        mesh=_mesh,
        scratch_types=[
            pltpu.VMEM((_CHUNK,), jnp.int32),
            pltpu.VMEM((_CHUNK,), jnp.int32),
            pltpu.VMEM((_CHUNK, width), jnp.float32),
            pltpu.VMEM_SHARED((_NP, width), jnp.float32),
            pltpu.SemaphoreType.DMA,
        ],
    )


_agg128 = _make_agg(_DH)
_agg64 = _make_agg(_DO)

_BLK = 1024
_GRID = _NP // _BLK


def _dinv_of(degp_ref):
    deg = degp_ref[0, :, 0:1] + degp_ref[1, :, 0:1] + 1.0
    return lax.rsqrt(deg)


def _h1_body(x_ref, w_ref, degp_ref, out_ref):
    h = jnp.dot(x_ref[...], w_ref[...], preferred_element_type=jnp.float32)
    out_ref[...] = h * _dinv_of(degp_ref)


_h1_call = pl.pallas_call(
    _h1_body,
    grid=(_GRID,),
    in_specs=[
        pl.BlockSpec((_BLK, _DH), lambda i: (i, 0)),
        pl.BlockSpec((_DH, _DH), lambda i: (0, 0)),
        pl.BlockSpec((_NC, _BLK, 16), lambda i: (0, i, 0)),
    ],
    out_specs=pl.BlockSpec((_BLK, _DH), lambda i: (i, 0)),
    out_shape=jax.ShapeDtypeStruct((_NP, _DH), jnp.float32),
)


def _h2_body(aggp_ref, h1s_ref, degp_ref, w2_ref, b1_ref, out_ref):
    dinv = _dinv_of(degp_ref)
    t = (aggp_ref[0] + aggp_ref[1] + h1s_ref[...]) * dinv + b1_ref[...]
    t = jnp.maximum(t, 0.0)
    out_ref[...] = jnp.dot(t, w2_ref[...], preferred_element_type=jnp.float32) * dinv


_h2_call = pl.pallas_call(
    _h2_body,
    grid=(_GRID,),
    in_specs=[
        pl.BlockSpec((_NC, _BLK, _DH), lambda i: (0, i, 0)),
        pl.BlockSpec((_BLK, _DH), lambda i: (i, 0)),
        pl.BlockSpec((_NC, _BLK, 16), lambda i: (0, i, 0)),
        pl.BlockSpec((_DH, _DO), lambda i: (0, 0)),
        pl.BlockSpec((1, _DH), lambda i: (0, 0)),
    ],
    out_specs=pl.BlockSpec((_BLK, _DO), lambda i: (i, 0)),
    out_shape=jax.ShapeDtypeStruct((_NP, _DO), jnp.float32),
)


def _out_body(aggp_ref, h2s_ref, degp_ref, b2_ref, out_ref):
    dinv = _dinv_of(degp_ref)
    o = (aggp_ref[0] + aggp_ref[1] + h2s_ref[...]) * dinv + b2_ref[...]
    col = lax.broadcasted_iota(jnp.int32, o.shape, 1)
    mask = col < _NCLS
    m = jnp.max(jnp.where(mask, o, -jnp.inf), axis=1, keepdims=True)
    sh = o - m
    se = jnp.sum(jnp.where(mask, jnp.exp(sh), 0.0), axis=1, keepdims=True)
    out_ref[...] = sh - jnp.log(se)


_out_call = pl.pallas_call(
    _out_body,
    grid=(_GRID,),
    in_specs=[
        pl.BlockSpec((_NC, _BLK, _DO), lambda i: (0, i, 0)),
        pl.BlockSpec((_BLK, _DO), lambda i: (i, 0)),
        pl.BlockSpec((_NC, _BLK, 16), lambda i: (0, i, 0)),
        pl.BlockSpec((1, _DO), lambda i: (0, 0)),
    ],
    out_specs=pl.BlockSpec((_BLK, _DO), lambda i: (i, 0)),
    out_shape=jax.ShapeDtypeStruct((_NP, _DO), jnp.float32),
)


def kernel(x, edge_index, W1, b1, W2, b2):
    f32 = jnp.float32
    x_pad = jnp.pad(x, ((0, _NP - _N), (0, 0)))
    src = edge_index[0]
    dst = edge_index[1]
    pad_e = _EP - _E
    src3 = jnp.concatenate(
        [src, jnp.zeros((pad_e,), src.dtype)]).reshape(_NW, _NCHUNK, _CHUNK)
    dst3 = jnp.concatenate(
        [dst, jnp.full((pad_e,), _N, dst.dtype)]).reshape(_NW, _NCHUNK, _CHUNK)
    ones16 = jnp.ones((_CHUNK, 16), f32)
    z16 = jnp.zeros((_RPT, 16), f32)
    z128 = jnp.zeros((_RPT, _DH), f32)
    z64 = jnp.zeros((_RPT, _DO), f32)

    degp = _deg_call(dst3, ones16, z16)
    h1s = _h1_call(x_pad, W1, degp)
    agg1p = _agg128(h1s, src3, dst3, z128)
    w2p = jnp.pad(W2, ((0, 0), (0, _DO - _NCLS)))
    h2s = _h2_call(agg1p, h1s, degp, w2p, b1.reshape(1, _DH))
    agg2p = _agg64(h2s, src3, dst3, z64)
    o = _out_call(agg2p, h2s, degp, jnp.pad(b2, (0, _DO - _NCLS)).reshape(1, _DO))
    return o[:_N, :_NCLS]


# speedup vs baseline: 8.2673x; 8.2673x over previous
import jax
import jax.numpy as jnp
from jax import lax
from jax.experimental import pallas as pl
from jax.experimental.pallas import tpu as pltpu
from jax.experimental.pallas import tpu_sc as plsc

_N = 10000
_NP = 10240
_E = 320000
_NC = 2
_NS = 16
_NW = _NC * _NS
_RPT = _NP // _NS
_CHUNK = 128
_NCHUNK = 80
_EP = _NW * _NCHUNK * _CHUNK
_DH = 128
_DO = 64
_NCLS = 40

_mesh = plsc.VectorSubcoreMesh(core_axis_name="c", subcore_axis_name="s")


def _deg_body(dst3, ones_hbm, zeros_hbm, out, didx_v, ones_v, deg_sp):
    c = lax.axis_index("c")
    s = lax.axis_index("s")
    wid = c * _NS + s
    r0 = s * _RPT
    pltpu.sync_copy(zeros_hbm, deg_sp.at[pl.ds(r0, _RPT)])
    pltpu.sync_copy(ones_hbm, ones_v)
    plsc.subcore_barrier()

    def step(j, carry):
        pltpu.sync_copy(dst3.at[wid, j], didx_v)
        pltpu.sync_copy(ones_v, deg_sp.at[didx_v], add=True)
        return carry

    lax.fori_loop(0, _NCHUNK, step, 0)
    plsc.subcore_barrier()
    pltpu.sync_copy(deg_sp.at[pl.ds(r0, _RPT)], out.at[c, pl.ds(r0, _RPT)])


_deg_call = pl.kernel(
    _deg_body,
    out_type=jax.ShapeDtypeStruct((_NC, _NP, 16), jnp.float32),
    mesh=_mesh,
    scratch_types=[
        pltpu.VMEM((_CHUNK,), jnp.int32),
        pltpu.VMEM((_CHUNK, 16), jnp.float32),
        pltpu.VMEM_SHARED((_NP, 16), jnp.float32),
    ],
)


def _make_agg(width):
    def body(h_hbm, src3, dst3, zeros_hbm, out, sidx_v, didx_v, rows_v, agg_sp, sem):
        c = lax.axis_index("c")
        s = lax.axis_index("s")
        wid = c * _NS + s
        r0 = s * _RPT
        pltpu.sync_copy(zeros_hbm, agg_sp.at[pl.ds(r0, _RPT)])
        plsc.subcore_barrier()

        def step(j, carry):
            pltpu.sync_copy(src3.at[wid, j], sidx_v)
            pltpu.async_copy(h_hbm.at[sidx_v], rows_v, sem).wait()
            pltpu.sync_copy(dst3.at[wid, j], didx_v)
            pltpu.sync_copy(rows_v, agg_sp.at[didx_v], add=True)
            return carry

        lax.fori_loop(0, _NCHUNK, step, 0)
        plsc.subcore_barrier()
        pltpu.sync_copy(agg_sp.at[pl.ds(r0, _RPT)], out.at[c, pl.ds(r0, _RPT)])

    return pl.kernel(
        body,
        out_type=jax.ShapeDtypeStruct((_NC, _NP, width), jnp.float32),
        mesh=_mesh,
        scratch_types=[
            pltpu.VMEM((_CHUNK,), jnp.int32),
            pltpu.VMEM((_CHUNK,), jnp.int32),
            pltpu.VMEM((_CHUNK, width), jnp.float32),
            pltpu.VMEM_SHARED((_NP, width), jnp.float32),
            pltpu.SemaphoreType.DMA,
        ],
    )


_agg128 = _make_agg(_DH)

_BLK = 1024
_GRID = _NP // _BLK


def _dinv_of(degp_ref):
    deg = degp_ref[0, :, 0:1] + degp_ref[1, :, 0:1] + 1.0
    return lax.rsqrt(deg)


def _h1_body(x_ref, w_ref, degp_ref, out_ref):
    h = jnp.dot(x_ref[...], w_ref[...], preferred_element_type=jnp.float32)
    out_ref[...] = h * _dinv_of(degp_ref)


_h1_call = pl.pallas_call(
    _h1_body,
    grid=(_GRID,),
    in_specs=[
        pl.BlockSpec((_BLK, _DH), lambda i: (i, 0)),
        pl.BlockSpec((_DH, _DH), lambda i: (0, 0)),
        pl.BlockSpec((_NC, _BLK, 16), lambda i: (0, i, 0)),
    ],
    out_specs=pl.BlockSpec((_BLK, _DH), lambda i: (i, 0)),
    out_shape=jax.ShapeDtypeStruct((_NP, _DH), jnp.float32),
)


def _h2_body(aggp_ref, h1s_ref, degp_ref, b1_ref, out_ref):
    dinv = _dinv_of(degp_ref)
    t = (aggp_ref[0] + aggp_ref[1] + h1s_ref[...]) * dinv + b1_ref[...]
    t = jnp.maximum(t, 0.0)
    out_ref[...] = t * dinv


_h2_call = pl.pallas_call(
    _h2_body,
    grid=(_GRID,),
    in_specs=[
        pl.BlockSpec((_NC, _BLK, _DH), lambda i: (0, i, 0)),
        pl.BlockSpec((_BLK, _DH), lambda i: (i, 0)),
        pl.BlockSpec((_NC, _BLK, 16), lambda i: (0, i, 0)),
        pl.BlockSpec((1, _DH), lambda i: (0, 0)),
    ],
    out_specs=pl.BlockSpec((_BLK, _DH), lambda i: (i, 0)),
    out_shape=jax.ShapeDtypeStruct((_NP, _DH), jnp.float32),
)


def _out_body(aggp_ref, rs_ref, degp_ref, w2_ref, b2_ref, out_ref):
    dinv = _dinv_of(degp_ref)
    t = (aggp_ref[0] + aggp_ref[1] + rs_ref[...]) * dinv
    o = jnp.dot(t, w2_ref[...], preferred_element_type=jnp.float32) + b2_ref[...]
    col = lax.broadcasted_iota(jnp.int32, o.shape, 1)
    mask = col < _NCLS
    m = jnp.max(jnp.where(mask, o, -jnp.inf), axis=1, keepdims=True)
    sh = o - m
    se = jnp.sum(jnp.where(mask, jnp.exp(sh), 0.0), axis=1, keepdims=True)
    out_ref[...] = sh - jnp.log(se)


_out_call = pl.pallas_call(
    _out_body,
    grid=(_GRID,),
    in_specs=[
        pl.BlockSpec((_NC, _BLK, _DH), lambda i: (0, i, 0)),
        pl.BlockSpec((_BLK, _DH), lambda i: (i, 0)),
        pl.BlockSpec((_NC, _BLK, 16), lambda i: (0, i, 0)),
        pl.BlockSpec((_DH, _DO), lambda i: (0, 0)),
        pl.BlockSpec((1, _DO), lambda i: (0, 0)),
    ],
    out_specs=pl.BlockSpec((_BLK, _DO), lambda i: (i, 0)),
    out_shape=jax.ShapeDtypeStruct((_NP, _DO), jnp.float32),
)


def kernel(x, edge_index, W1, b1, W2, b2):
    f32 = jnp.float32
    x_pad = jnp.pad(x, ((0, _NP - _N), (0, 0)))
    src = edge_index[0]
    dst = edge_index[1]
    pad_e = _EP - _E
    src3 = jnp.concatenate(
        [src, jnp.zeros((pad_e,), src.dtype)]).reshape(_NW, _NCHUNK, _CHUNK)
    dst3 = jnp.concatenate(
        [dst, jnp.full((pad_e,), _N, dst.dtype)]).reshape(_NW, _NCHUNK, _CHUNK)
    ones16 = jnp.ones((_CHUNK, 16), f32)
    z16 = jnp.zeros((_RPT, 16), f32)
    z128 = jnp.zeros((_RPT, _DH), f32)

    degp = _deg_call(dst3, ones16, z16)
    h1s = _h1_call(x_pad, W1, degp)
    agg1p = _agg128(h1s, src3, dst3, z128)
    rs = _h2_call(agg1p, h1s, degp, b1.reshape(1, _DH))
    agg2p = _agg128(rs, src3, dst3, z128)
    w2p = jnp.pad(W2, ((0, 0), (0, _DO - _NCLS)))
    o = _out_call(agg2p, rs, degp, w2p, jnp.pad(b2, (0, _DO - _NCLS)).reshape(1, _DO))
    return o[:_N, :_NCLS]


# speedup vs baseline: 10.3391x; 1.2506x over previous
import jax
import jax.numpy as jnp
from jax import lax
from jax.experimental import pallas as pl
from jax.experimental.pallas import tpu as pltpu
from jax.experimental.pallas import tpu_sc as plsc

_N = 10000
_NP = 10240
_E = 320000
_NC = 2
_NS = 16
_NW = _NC * _NS
_RPT = _NP // _NS
_CHUNK = 128
_NCHUNK = 80
_EP = _NW * _NCHUNK * _CHUNK
_DH = 128
_DO = 64
_NCLS = 40

_mesh = plsc.VectorSubcoreMesh(core_axis_name="c", subcore_axis_name="s")


def _deg_body(dst3, ones_hbm, zeros_hbm, out, didx_v, ones_v, deg_sp):
    c = lax.axis_index("c")
    s = lax.axis_index("s")
    wid = c * _NS + s
    r0 = s * _RPT
    pltpu.sync_copy(zeros_hbm, deg_sp.at[pl.ds(r0, _RPT)])
    pltpu.sync_copy(ones_hbm, ones_v)
    plsc.subcore_barrier()

    def step(j, carry):
        pltpu.sync_copy(dst3.at[wid, j], didx_v)
        pltpu.sync_copy(ones_v, deg_sp.at[didx_v], add=True)
        return carry

    lax.fori_loop(0, _NCHUNK, step, 0)
    plsc.subcore_barrier()
    pltpu.sync_copy(deg_sp.at[pl.ds(r0, _RPT)], out.at[c, pl.ds(r0, _RPT)])


_deg_call = pl.kernel(
    _deg_body,
    out_type=jax.ShapeDtypeStruct((_NC, _NP, 16), jnp.float32),
    mesh=_mesh,
    scratch_types=[
        pltpu.VMEM((_CHUNK,), jnp.int32),
        pltpu.VMEM((_CHUNK, 16), jnp.float32),
        pltpu.VMEM_SHARED((_NP, 16), jnp.float32),
    ],
)

_NBUF = 2
_GRP = 8
_NGRP = _NCHUNK // _GRP


def _make_agg(width):
    def body(h_hbm, src3, dst3, zeros_hbm, out, sidx0, sidx1, didx_v, rows_v,
             agg_sp, sem0, sem1):
        sems = [sem0, sem1]
        sidxs = [sidx0, sidx1]
        c = lax.axis_index("c")
        s = lax.axis_index("s")
        wid = c * _NS + s
        r0 = s * _RPT
        pltpu.sync_copy(zeros_hbm, agg_sp.at[pl.ds(r0, _RPT)])
        plsc.subcore_barrier()

        for b in range(_NBUF):
            pltpu.sync_copy(src3.at[wid, b], sidxs[b])
            pltpu.async_copy(h_hbm.at[sidxs[b]], rows_v.at[b], sems[b])

        def group(g, carry):
            for b in range(_NBUF):
                j = g * _NBUF + b
                pltpu.make_async_copy(h_hbm.at[sidxs[b]], rows_v.at[b],
                                      sems[b]).wait()
                pltpu.sync_copy(dst3.at[wid, j], didx_v)
                pltpu.sync_copy(rows_v.at[b], agg_sp.at[didx_v], add=True)

                @pl.when(j + _NBUF < _NCHUNK)
                def _():
                    pltpu.sync_copy(src3.at[wid, j + _NBUF], sidxs[b])
                    pltpu.async_copy(h_hbm.at[sidxs[b]], rows_v.at[b], sems[b])
            return carry

        lax.fori_loop(0, _NCHUNK // _NBUF, group, 0)
        plsc.subcore_barrier()
        pltpu.sync_copy(agg_sp.at[pl.ds(r0, _RPT)], out.at[c, pl.ds(r0, _RPT)])

    return pl.kernel(
        body,
        out_type=jax.ShapeDtypeStruct((_NC, _NP, width), jnp.float32),
        mesh=_mesh,
        scratch_types=[
            pltpu.VMEM((_CHUNK,), jnp.int32),
            pltpu.VMEM((_CHUNK,), jnp.int32),
            pltpu.VMEM((_CHUNK,), jnp.int32),
            pltpu.VMEM((_NBUF, _CHUNK, width), jnp.float32),
            pltpu.VMEM_SHARED((_NP, width), jnp.float32),
            pltpu.SemaphoreType.DMA,
            pltpu.SemaphoreType.DMA,
        ],
    )


_agg128 = _make_agg(_DH)

_BLK = 1024
_GRID = _NP // _BLK


def _dinv_of(degp_ref):
    deg = degp_ref[0, :, 0:1] + degp_ref[1, :, 0:1] + 1.0
    return lax.rsqrt(deg)


def _h1_body(x_ref, w_ref, degp_ref, out_ref):
    h = jnp.dot(x_ref[...], w_ref[...], preferred_element_type=jnp.float32)
    out_ref[...] = h * _dinv_of(degp_ref)


_h1_call = pl.pallas_call(
    _h1_body,
    grid=(_GRID,),
    in_specs=[
        pl.BlockSpec((_BLK, _DH), lambda i: (i, 0)),
        pl.BlockSpec((_DH, _DH), lambda i: (0, 0)),
        pl.BlockSpec((_NC, _BLK, 16), lambda i: (0, i, 0)),
    ],
    out_specs=pl.BlockSpec((_BLK, _DH), lambda i: (i, 0)),
    out_shape=jax.ShapeDtypeStruct((_NP, _DH), jnp.float32),
)


def _h2_body(aggp_ref, h1s_ref, degp_ref, b1_ref, out_ref):
    dinv = _dinv_of(degp_ref)
    t = (aggp_ref[0] + aggp_ref[1] + h1s_ref[...]) * dinv + b1_ref[...]
    t = jnp.maximum(t, 0.0)
    out_ref[...] = t * dinv


_h2_call = pl.pallas_call(
    _h2_body,
    grid=(_GRID,),
    in_specs=[
        pl.BlockSpec((_NC, _BLK, _DH), lambda i: (0, i, 0)),
        pl.BlockSpec((_BLK, _DH), lambda i: (i, 0)),
        pl.BlockSpec((_NC, _BLK, 16), lambda i: (0, i, 0)),
        pl.BlockSpec((1, _DH), lambda i: (0, 0)),
    ],
    out_specs=pl.BlockSpec((_BLK, _DH), lambda i: (i, 0)),
    out_shape=jax.ShapeDtypeStruct((_NP, _DH), jnp.float32),
)


def _out_body(aggp_ref, rs_ref, degp_ref, w2_ref, b2_ref, out_ref):
    dinv = _dinv_of(degp_ref)
    t = (aggp_ref[0] + aggp_ref[1] + rs_ref[...]) * dinv
    o = jnp.dot(t, w2_ref[...], preferred_element_type=jnp.float32) + b2_ref[...]
    col = lax.broadcasted_iota(jnp.int32, o.shape, 1)
    mask = col < _NCLS
    m = jnp.max(jnp.where(mask, o, -jnp.inf), axis=1, keepdims=True)
    sh = o - m
    se = jnp.sum(jnp.where(mask, jnp.exp(sh), 0.0), axis=1, keepdims=True)
    out_ref[...] = sh - jnp.log(se)


_out_call = pl.pallas_call(
    _out_body,
    grid=(_GRID,),
    in_specs=[
        pl.BlockSpec((_NC, _BLK, _DH), lambda i: (0, i, 0)),
        pl.BlockSpec((_BLK, _DH), lambda i: (i, 0)),
        pl.BlockSpec((_NC, _BLK, 16), lambda i: (0, i, 0)),
        pl.BlockSpec((_DH, _DO), lambda i: (0, 0)),
        pl.BlockSpec((1, _DO), lambda i: (0, 0)),
    ],
    out_specs=pl.BlockSpec((_BLK, _DO), lambda i: (i, 0)),
    out_shape=jax.ShapeDtypeStruct((_NP, _DO), jnp.float32),
)


def kernel(x, edge_index, W1, b1, W2, b2):
    f32 = jnp.float32
    x_pad = jnp.pad(x, ((0, _NP - _N), (0, 0)))
    src = edge_index[0]
    dst = edge_index[1]
    pad_e = _EP - _E
    src3 = jnp.concatenate(
        [src, jnp.zeros((pad_e,), src.dtype)]).reshape(_NW, _NCHUNK, _CHUNK)
    dst3 = jnp.concatenate(
        [dst, jnp.full((pad_e,), _N, dst.dtype)]).reshape(_NW, _NCHUNK, _CHUNK)
    ones16 = jnp.ones((_CHUNK, 16), f32)
    z16 = jnp.zeros((_RPT, 16), f32)
    z128 = jnp.zeros((_RPT, _DH), f32)

    degp = _deg_call(dst3, ones16, z16)
    h1s = _h1_call(x_pad, W1, degp)
    agg1p = _agg128(h1s, src3, dst3, z128)
    rs = _h2_call(agg1p, h1s, degp, b1.reshape(1, _DH))
    agg2p = _agg128(rs, src3, dst3, z128)
    w2p = jnp.pad(W2, ((0, 0), (0, _DO - _NCLS)))
    o = _out_call(agg2p, rs, degp, w2p, jnp.pad(b2, (0, _DO - _NCLS)).reshape(1, _DO))
    return o[:_N, :_NCLS]
